# Initial kernel scaffold; baseline (speedup 1.0000x reference)
import jax
import jax.numpy as jnp
from jax.experimental import pallas as pl


def kernel(x_tokens, w1, b1, gn1_w, gn1_b, wd, bd, gn2_w, gn2_b, w2, b2, gn3_w, gn3_b, ln_w, ln_b, w_red):
    raise NotImplementedError("write your pallas kernel here")


# speedup vs baseline: 16.5499x; 16.5499x over previous
import functools

import jax
import jax.numpy as jnp
from jax.experimental import pallas as pl
from jax.experimental.pallas import tpu as pltpu

_EPS = 1e-5


def _group_sum_matrix(C, gc):
    r = jax.lax.broadcasted_iota(jnp.int32, (C, C), 0) // gc
    c = jax.lax.broadcasted_iota(jnp.int32, (C, C), 1) // gc
    return (r == c).astype(jnp.float32)


def _gn(h, gamma, beta, gc, gelu):
    C, N = h.shape
    A = _group_sum_matrix(C, gc)
    s = jnp.sum(h, axis=1, keepdims=True)
    s2 = jnp.sum(h * h, axis=1, keepdims=True)
    gs = jnp.dot(A, s, preferred_element_type=jnp.float32)
    gs2 = jnp.dot(A, s2, preferred_element_type=jnp.float32)
    cnt = gc * N
    mu = gs / cnt
    var = gs2 / cnt - mu * mu
    rstd = jax.lax.rsqrt(var + _EPS)
    y = (h - mu) * (rstd * gamma) + beta
    if gelu:
        y = jax.nn.gelu(y, approximate=True)
    return y


def _dwconv_quad(h, wtap_ref, bd):
    blocks = [h[:, k * 1024:(k + 1) * 1024] for k in range(4)]
    lane = jax.lax.broadcasted_iota(jnp.int32, (1, 1024), 1)
    i_idx = lane // 32
    j_idx = lane % 32

    outs = []
    for k in range(4):
        hp, wp = k % 2, k // 2
        acc = jnp.zeros((32, 1024), jnp.float32)
        for oy in (-1, 0, 1):
            hp2 = (hp + oy) % 2
            di = (hp + oy) // 2
            for ox in (-1, 0, 1):
                wp2 = (wp + ox) % 2
                dj = (wp + ox) // 2
                src = blocks[2 * wp2 + hp2]
                s = di * 32 + dj
                v = jnp.roll(src, -s, axis=1) if s != 0 else src
                if di != 0 or dj != 0:
                    m = jnp.ones((1, 1024), jnp.bool_)
                    if di != 0:
                        m = m & (i_idx + di >= 0) & (i_idx + di < 32)
                    if dj != 0:
                        m = m & (j_idx + dj >= 0) & (j_idx + dj < 32)
                    v = jnp.where(m, v, 0.0)
                t = (oy + 1) * 3 + (ox + 1)
                acc = acc + jnp.dot(wtap_ref[t], v,
                                    preferred_element_type=jnp.float32)
        outs.append(acc)
    return jnp.concatenate(outs, axis=1) + bd


def _fused_kernel(x_ref, w1_ref, b1_ref, g1w_ref, g1b_ref, wtap_ref, bd_ref,
                  g2w_ref, g2b_ref, w2_ref, b2_ref, g3w_ref, g3b_ref,
                  lnw_ref, lnb_ref, wred_ref, o_ref):
    x = x_ref[0]

    h = jnp.dot(w1_ref[...], x, preferred_element_type=jnp.float32) + b1_ref[...]
    h = _gn(h, g1w_ref[...], g1b_ref[...], 4, gelu=False)

    h = _dwconv_quad(h, wtap_ref, bd_ref[...])
    h = _gn(h, g2w_ref[...], g2b_ref[...], 4, gelu=True)

    o = jnp.dot(w2_ref[...], h, preferred_element_type=jnp.float32) + b2_ref[...]
    o = _gn(o, g3w_ref[...], g3b_ref[...], 4, gelu=False)

    t = jnp.concatenate([o[:, 0:1024], o[:, 1024:2048],
                         o[:, 2048:3072], o[:, 3072:4096]], axis=0)

    mu = jnp.mean(t, axis=0, keepdims=True)
    var = jnp.mean(t * t, axis=0, keepdims=True) - mu * mu
    tn = (t - mu) * jax.lax.rsqrt(var + _EPS)
    tn = tn * lnw_ref[...] + lnb_ref[...]

    o_ref[0] = jnp.dot(wred_ref[...], tn, preferred_element_type=jnp.float32)


def kernel(x_tokens, w1, b1, gn1_w, gn1_b, wd, bd, gn2_w, gn2_b,
           w2, b2, gn3_w, gn3_b, ln_w, ln_b, w_red):
    B, N, Cin = x_tokens.shape
    H = W = 64
    Ch = w1.shape[0]
    Cout = w2.shape[0]
    C4 = 4 * Cout
    Cred = w_red.shape[0]
    N4 = N // 4

    f32 = jnp.float32

    xq = x_tokens.reshape(B, 32, 2, 32, 2, Cin).transpose(0, 5, 4, 2, 1, 3)
    xq = xq.reshape(B, Cin, N).astype(f32)

    G = Ch // 4
    wd_r = wd.astype(f32).reshape(G, 4, 4, 3, 3)
    wd_t = jnp.transpose(wd_r, (3, 4, 0, 1, 2))
    eye = jnp.eye(G, dtype=f32)
    w9 = (wd_t[:, :, :, :, None, :] *
          eye[None, None, :, None, :, None]).reshape(9, Ch, Ch)

    col = lambda v, C: v.astype(f32).reshape(C, 1)

    out = pl.pallas_call(
        _fused_kernel,
        out_shape=jax.ShapeDtypeStruct((B, Cred, N4), f32),
        grid_spec=pltpu.PrefetchScalarGridSpec(
            num_scalar_prefetch=0,
            grid=(B,),
            in_specs=[
                pl.BlockSpec((1, Cin, N), lambda b: (b, 0, 0)),
                pl.BlockSpec((Ch, Cin), lambda b: (0, 0)),
                pl.BlockSpec((Ch, 1), lambda b: (0, 0)),
                pl.BlockSpec((Ch, 1), lambda b: (0, 0)),
                pl.BlockSpec((Ch, 1), lambda b: (0, 0)),
                pl.BlockSpec((9, Ch, Ch), lambda b: (0, 0, 0)),
                pl.BlockSpec((Ch, 1), lambda b: (0, 0)),
                pl.BlockSpec((Ch, 1), lambda b: (0, 0)),
                pl.BlockSpec((Ch, 1), lambda b: (0, 0)),
                pl.BlockSpec((Cout, Ch), lambda b: (0, 0)),
                pl.BlockSpec((Cout, 1), lambda b: (0, 0)),
                pl.BlockSpec((Cout, 1), lambda b: (0, 0)),
                pl.BlockSpec((Cout, 1), lambda b: (0, 0)),
                pl.BlockSpec((C4, 1), lambda b: (0, 0)),
                pl.BlockSpec((C4, 1), lambda b: (0, 0)),
                pl.BlockSpec((Cred, C4), lambda b: (0, 0)),
            ],
            out_specs=pl.BlockSpec((1, Cred, N4), lambda b: (b, 0, 0)),
        ),
        compiler_params=pltpu.CompilerParams(
            dimension_semantics=("parallel",)),
        cost_estimate=pl.CostEstimate(
            flops=2 * B * N * (Ch * Cin + Ch * Ch * 9 // 4 + Cout * Ch)
                  + 2 * B * N4 * C4 * Cred + 20 * B * N * (Ch + Cout),
            transcendentals=B * Ch * N,
            bytes_accessed=4 * (B * Cin * N + B * Cred * N4
                                + Ch * Cin + Cout * Ch + Cred * C4)),
    )(xq,
      w1.astype(f32), col(b1, Ch), col(gn1_w, Ch), col(gn1_b, Ch),
      w9, col(bd, Ch), col(gn2_w, Ch), col(gn2_b, Ch),
      w2.astype(f32), col(b2, Cout), col(gn3_w, Cout), col(gn3_b, Cout),
      col(ln_w, C4), col(ln_b, C4),
      w_red.astype(f32))

    return out.reshape(B, Cred, H // 2, W // 2)


# speedup vs baseline: 21.5100x; 1.2997x over previous
import functools

import jax
import jax.numpy as jnp
from jax.experimental import pallas as pl
from jax.experimental.pallas import tpu as pltpu

_EPS = 1e-5


def _group_sum_matrix(C, gc):
    r = jax.lax.broadcasted_iota(jnp.int32, (C, C), 0) // gc
    c = jax.lax.broadcasted_iota(jnp.int32, (C, C), 1) // gc
    return (r == c).astype(jnp.float32)


def _gn(h, gamma, beta, gc, gelu):
    C, N = h.shape
    A = _group_sum_matrix(C, gc)
    s = jnp.sum(h, axis=1, keepdims=True)
    s2 = jnp.sum(h * h, axis=1, keepdims=True)
    gs = jnp.dot(A, s, preferred_element_type=jnp.float32)
    gs2 = jnp.dot(A, s2, preferred_element_type=jnp.float32)
    cnt = gc * N
    mu = gs / cnt
    var = gs2 / cnt - mu * mu
    rstd = jax.lax.rsqrt(var + _EPS)
    y = (h - mu) * (rstd * gamma) + beta
    if gelu:
        y = jax.nn.gelu(y, approximate=True)
    return y


def _dwconv_quad(h, wtap_ref, bd):
    blocks = [h[:, k * 1024:(k + 1) * 1024] for k in range(4)]
    lane = jax.lax.broadcasted_iota(jnp.int32, (1, 1024), 1)
    i_idx = lane // 32
    j_idx = lane % 32

    outs = []
    for k in range(4):
        hp, wp = k % 2, k // 2
        acc = jnp.zeros((32, 1024), jnp.float32)
        for oy in (-1, 0, 1):
            hp2 = (hp + oy) % 2
            di = (hp + oy) // 2
            for ox in (-1, 0, 1):
                wp2 = (wp + ox) % 2
                dj = (wp + ox) // 2
                src = blocks[2 * wp2 + hp2]
                s = di * 32 + dj
                v = jnp.roll(src, -s, axis=1) if s != 0 else src
                if di != 0 or dj != 0:
                    m = jnp.ones((1, 1024), jnp.bool_)
                    if di != 0:
                        m = m & (i_idx + di >= 0) & (i_idx + di < 32)
                    if dj != 0:
                        m = m & (j_idx + dj >= 0) & (j_idx + dj < 32)
                    v = jnp.where(m, v, 0.0)
                t = (oy + 1) * 3 + (ox + 1)
                acc = acc + jnp.dot(wtap_ref[t], v,
                                    preferred_element_type=jnp.float32)
        outs.append(acc)
    return jnp.concatenate(outs, axis=1) + bd


def _fused_kernel(xh0_ref, xh1_ref,
                  w1_ref, b1_ref, g1w_ref, g1b_ref, wtap_ref, bd_ref,
                  g2w_ref, g2b_ref, w2_ref, b2_ref, g3w_ref, g3b_ref,
                  lnw_ref, lnb_ref, wred_ref, o_ref):
    dims = (((1,), (1,)), ((), ()))
    xh = [xh0_ref[0].reshape(1024, 256), xh1_ref[0].reshape(1024, 256)]
    hs = []
    for k in range(4):
        hp, wp = k % 2, k // 2
        xk = xh[hp][:, wp * 128:(wp + 1) * 128]
        hs.append(jax.lax.dot_general(w1_ref[...], xk, dims,
                                      preferred_element_type=jnp.float32))
    h = jnp.concatenate(hs, axis=1) + b1_ref[...]
    h = _gn(h, g1w_ref[...], g1b_ref[...], 4, gelu=False)

    h = _dwconv_quad(h, wtap_ref, bd_ref[...])
    h = _gn(h, g2w_ref[...], g2b_ref[...], 4, gelu=True)

    o = jnp.dot(w2_ref[...], h, preferred_element_type=jnp.float32) + b2_ref[...]
    o = _gn(o, g3w_ref[...], g3b_ref[...], 4, gelu=False)

    t = jnp.concatenate([o[:, 0:1024], o[:, 1024:2048],
                         o[:, 2048:3072], o[:, 3072:4096]], axis=0)

    mu = jnp.mean(t, axis=0, keepdims=True)
    var = jnp.mean(t * t, axis=0, keepdims=True) - mu * mu
    tn = (t - mu) * jax.lax.rsqrt(var + _EPS)
    tn = tn * lnw_ref[...] + lnb_ref[...]

    o_ref[0] = jnp.dot(wred_ref[...], tn, preferred_element_type=jnp.float32)


def kernel(x_tokens, w1, b1, gn1_w, gn1_b, wd, bd, gn2_w, gn2_b,
           w2, b2, gn3_w, gn3_b, ln_w, ln_b, w_red):
    B, N, Cin = x_tokens.shape
    H = W = 64
    Ch = w1.shape[0]
    Cout = w2.shape[0]
    C4 = 4 * Cout
    Cred = w_red.shape[0]
    N4 = N // 4

    f32 = jnp.float32

    xr = x_tokens.astype(f32).reshape(B, 32, 2, 32, 2 * Cin)

    G = Ch // 4
    wd_r = wd.astype(f32).reshape(G, 4, 4, 3, 3)
    wd_t = jnp.transpose(wd_r, (3, 4, 0, 1, 2))
    eye = jnp.eye(G, dtype=f32)
    w9 = (wd_t[:, :, :, :, None, :] *
          eye[None, None, :, None, :, None]).reshape(9, Ch, Ch)

    col = lambda v, C: v.astype(f32).reshape(C, 1)

    out = pl.pallas_call(
        _fused_kernel,
        out_shape=jax.ShapeDtypeStruct((B, Cred, N4), f32),
        grid_spec=pltpu.PrefetchScalarGridSpec(
            num_scalar_prefetch=0,
            grid=(B,),
            in_specs=[
                pl.BlockSpec((1, 32, None, 32, 2 * Cin),
                             lambda b: (b, 0, 0, 0, 0)),
                pl.BlockSpec((1, 32, None, 32, 2 * Cin),
                             lambda b: (b, 0, 1, 0, 0)),
                pl.BlockSpec((Ch, Cin), lambda b: (0, 0)),
                pl.BlockSpec((Ch, 1), lambda b: (0, 0)),
                pl.BlockSpec((Ch, 1), lambda b: (0, 0)),
                pl.BlockSpec((Ch, 1), lambda b: (0, 0)),
                pl.BlockSpec((9, Ch, Ch), lambda b: (0, 0, 0)),
                pl.BlockSpec((Ch, 1), lambda b: (0, 0)),
                pl.BlockSpec((Ch, 1), lambda b: (0, 0)),
                pl.BlockSpec((Ch, 1), lambda b: (0, 0)),
                pl.BlockSpec((Cout, Ch), lambda b: (0, 0)),
                pl.BlockSpec((Cout, 1), lambda b: (0, 0)),
                pl.BlockSpec((Cout, 1), lambda b: (0, 0)),
                pl.BlockSpec((Cout, 1), lambda b: (0, 0)),
                pl.BlockSpec((C4, 1), lambda b: (0, 0)),
                pl.BlockSpec((C4, 1), lambda b: (0, 0)),
                pl.BlockSpec((Cred, C4), lambda b: (0, 0)),
            ],
            out_specs=pl.BlockSpec((1, Cred, N4), lambda b: (b, 0, 0)),
        ),
        compiler_params=pltpu.CompilerParams(
            dimension_semantics=("parallel",)),
        cost_estimate=pl.CostEstimate(
            flops=2 * B * N * (Ch * Cin + Ch * Ch * 9 // 4 + Cout * Ch)
                  + 2 * B * N4 * C4 * Cred + 20 * B * N * (Ch + Cout),
            transcendentals=B * Ch * N,
            bytes_accessed=4 * (B * Cin * N + B * Cred * N4
                                + Ch * Cin + Cout * Ch + Cred * C4)),
    )(xr, xr,
      w1.astype(f32), col(b1, Ch), col(gn1_w, Ch), col(gn1_b, Ch),
      w9, col(bd, Ch), col(gn2_w, Ch), col(gn2_b, Ch),
      w2.astype(f32), col(b2, Cout), col(gn3_w, Cout), col(gn3_b, Cout),
      col(ln_w, C4), col(ln_b, C4),
      w_red.astype(f32))

    return out.reshape(B, Cred, H // 2, W // 2)
